# baseline (speedup 1.0000x reference)
import jax
import jax.numpy as jnp
from jax import lax
from jax.experimental import pallas as pl
from jax.experimental.pallas import tpu as pltpu

_TB = 1024
_HT = 128


def _fk(inds_ref,
        state_ref,
        mat_l_ref, mat_r_ref,
        goal_ref, wg_ref, bg_ref,
        ws_ref, bs_ref,
        sse_ref,
        w_ref,
        bvec_ref,
        counts_ref,
        accv_ref):
    t = pl.program_id(0)
    nsteps = pl.num_programs(0)
    nhist = (nsteps * _TB) // _HT

    @pl.when(t == 0)
    def _init():
        accv_ref[...] = jnp.zeros_like(accv_ref)
        g = jnp.dot(goal_ref[...].astype(jnp.bfloat16),
                    wg_ref[...].astype(jnp.bfloat16),
                    preferred_element_type=jnp.float32)
        goal_out = (g + bg_ref[...]).astype(jnp.bfloat16)
        w = lax.dot_general(ws_ref[...].astype(jnp.bfloat16), goal_out,
                            (((1,), (1,)), ((), ())),
                            preferred_element_type=jnp.float32)
        w_ref[...] = w.astype(jnp.bfloat16)
        bvec_ref[...] = lax.dot_general(
            bs_ref[...].astype(jnp.bfloat16), goal_out,
            (((1,), (1,)), ((), ())),
            preferred_element_type=jnp.float32)
        idx = inds_ref[...]
        hi = idx // _HT
        lo = idx - hi * _HT
        hi_t = lax.broadcasted_iota(jnp.int32, (nhist, 1), 0)
        lo_t = lax.broadcasted_iota(jnp.int32, (_HT, 1), 0)
        hi_oh = (hi == hi_t).astype(jnp.bfloat16)
        lo_oh = (lo == lo_t).astype(jnp.bfloat16)
        counts = lax.dot_general(
            hi_oh, lo_oh, (((1,), (1,)), ((), ())),
            preferred_element_type=jnp.float32)
        counts_ref[...] = counts.reshape(counts_ref.shape)

    s_bf = state_ref[...].astype(jnp.bfloat16)
    w_row = counts_ref[pl.ds(t, 1), :]
    half = mat_l_ref.shape[1]
    for h, mref in enumerate((mat_l_ref, mat_r_ref)):
        lo_c, hi_c = h * half, (h + 1) * half
        recon = jnp.dot(s_bf, w_ref[:, lo_c:hi_c],
                        preferred_element_type=jnp.float32) + bvec_ref[:, lo_c:hi_c]
        diff = recon - mref[...]
        accv_ref[:, lo_c:hi_c] += jnp.dot(w_row, diff * diff,
                                          preferred_element_type=jnp.float32)

    @pl.when(t == nsteps - 1)
    def _finalize():
        sse_ref[0, 0] = jnp.sum(accv_ref[...])


@jax.jit
def kernel(inds, state_inp, goal_inp, sparse_value_mat, ws, bs, wg, bg):
    M, Ds = state_inp.shape
    N, Dg = goal_inp.shape
    H = ws.shape[1]
    B = inds.shape[0]

    inds2 = inds.astype(jnp.int32).reshape(1, B)
    bs2 = bs.reshape(1, H).astype(jnp.float32)
    bg2 = bg.reshape(1, H).astype(jnp.float32)

    num_tiles = M // _TB

    sse = pl.pallas_call(
        _fk,
        grid=(num_tiles,),
        in_specs=[
            pl.BlockSpec((1, B), lambda t: (0, 0)),
            pl.BlockSpec((_TB, Ds), lambda t: (t, 0)),
            pl.BlockSpec((_TB, N // 2), lambda t: (t, 0)),
            pl.BlockSpec((_TB, N // 2), lambda t: (t, 1)),
            pl.BlockSpec((N, Dg), lambda t: (0, 0)),
            pl.BlockSpec((Dg, H), lambda t: (0, 0)),
            pl.BlockSpec((1, H), lambda t: (0, 0)),
            pl.BlockSpec((Ds, H), lambda t: (0, 0)),
            pl.BlockSpec((1, H), lambda t: (0, 0)),
        ],
        out_specs=pl.BlockSpec(memory_space=pltpu.MemorySpace.SMEM),
        out_shape=jax.ShapeDtypeStruct((1, 1), jnp.float32),
        scratch_shapes=[
            pltpu.VMEM((Ds, N), jnp.bfloat16),
            pltpu.VMEM((1, N), jnp.float32),
            pltpu.VMEM((M // _TB, _TB), jnp.float32),
            pltpu.VMEM((1, N), jnp.float32),
        ],
        compiler_params=pltpu.CompilerParams(
            dimension_semantics=("arbitrary",),
            vmem_limit_bytes=64 * 1024 * 1024,
        ),
        cost_estimate=pl.CostEstimate(
            flops=2 * N * Dg * H + 2 * Ds * H * N + 2 * M * Ds * N + 3 * M * N,
            transcendentals=0,
            bytes_accessed=(M * Ds * 4 + M * N * 4 + N * Dg * 4
                            + Dg * H * 4 + Ds * H * 4 + 2 * H * 4 + B * 4),
        ),
    )(inds2, state_inp, sparse_value_mat, sparse_value_mat,
      goal_inp, wg, bg2, ws, bs2)

    return sse[0, 0]


# speedup vs baseline: 1.0020x; 1.0020x over previous
import jax
import jax.numpy as jnp
from jax import lax
from jax.experimental import pallas as pl
from jax.experimental.pallas import tpu as pltpu

_TB = 2048
_HT = 128


def _fk(inds_ref,
        state_ref,
        mat_ref,
        goal_ref, wg_ref, bg_ref,
        ws_ref, bs_ref,
        sse_ref,
        w_ref,
        bvec_ref,
        counts_ref,
        accv_ref):
    t = pl.program_id(0)
    nsteps = pl.num_programs(0)
    nhist = (nsteps * _TB) // _HT

    @pl.when(t == 0)
    def _init():
        accv_ref[...] = jnp.zeros_like(accv_ref)
        g = jnp.dot(goal_ref[...].astype(jnp.bfloat16),
                    wg_ref[...].astype(jnp.bfloat16),
                    preferred_element_type=jnp.float32)
        goal_out = (g + bg_ref[...]).astype(jnp.bfloat16)
        w = lax.dot_general(ws_ref[...].astype(jnp.bfloat16), goal_out,
                            (((1,), (1,)), ((), ())),
                            preferred_element_type=jnp.float32)
        w_ref[...] = w.astype(jnp.bfloat16)
        bvec_ref[...] = lax.dot_general(
            bs_ref[...].astype(jnp.bfloat16), goal_out,
            (((1,), (1,)), ((), ())),
            preferred_element_type=jnp.float32)
        idx = inds_ref[...]
        hi = idx // _HT
        lo = idx - hi * _HT
        hi_t = lax.broadcasted_iota(jnp.int32, (nhist, 1), 0)
        lo_t = lax.broadcasted_iota(jnp.int32, (_HT, 1), 0)
        hi_oh = (hi == hi_t).astype(jnp.bfloat16)
        lo_oh = (lo == lo_t).astype(jnp.bfloat16)
        counts = lax.dot_general(
            hi_oh, lo_oh, (((1,), (1,)), ((), ())),
            preferred_element_type=jnp.float32)
        counts_ref[...] = counts.reshape(counts_ref.shape)

    recon = jnp.dot(state_ref[...].astype(jnp.bfloat16), w_ref[...],
                    preferred_element_type=jnp.float32) + bvec_ref[...]
    diff = recon - mat_ref[...]
    sq = diff * diff
    w_row = counts_ref[pl.ds(t, 1), :]
    accv_ref[...] += jnp.dot(w_row, sq,
                             preferred_element_type=jnp.float32)

    @pl.when(t == nsteps - 1)
    def _finalize():
        sse_ref[0, 0] = jnp.sum(accv_ref[...])


@jax.jit
def kernel(inds, state_inp, goal_inp, sparse_value_mat, ws, bs, wg, bg):
    M, Ds = state_inp.shape
    N, Dg = goal_inp.shape
    H = ws.shape[1]
    B = inds.shape[0]

    inds2 = inds.astype(jnp.int32).reshape(1, B)
    bs2 = bs.reshape(1, H).astype(jnp.float32)
    bg2 = bg.reshape(1, H).astype(jnp.float32)

    num_tiles = M // _TB

    sse = pl.pallas_call(
        _fk,
        grid=(num_tiles,),
        in_specs=[
            pl.BlockSpec((1, B), lambda t: (0, 0)),
            pl.BlockSpec((_TB, Ds), lambda t: (t, 0)),
            pl.BlockSpec((_TB, N), lambda t: (t, 0)),
            pl.BlockSpec((N, Dg), lambda t: (0, 0)),
            pl.BlockSpec((Dg, H), lambda t: (0, 0)),
            pl.BlockSpec((1, H), lambda t: (0, 0)),
            pl.BlockSpec((Ds, H), lambda t: (0, 0)),
            pl.BlockSpec((1, H), lambda t: (0, 0)),
        ],
        out_specs=pl.BlockSpec(memory_space=pltpu.MemorySpace.SMEM),
        out_shape=jax.ShapeDtypeStruct((1, 1), jnp.float32),
        scratch_shapes=[
            pltpu.VMEM((Ds, N), jnp.bfloat16),
            pltpu.VMEM((1, N), jnp.float32),
            pltpu.VMEM((M // _TB, _TB), jnp.float32),
            pltpu.VMEM((1, N), jnp.float32),
        ],
        compiler_params=pltpu.CompilerParams(
            dimension_semantics=("arbitrary",),
            vmem_limit_bytes=64 * 1024 * 1024,
        ),
        cost_estimate=pl.CostEstimate(
            flops=2 * N * Dg * H + 2 * Ds * H * N + 2 * M * Ds * N + 3 * M * N,
            transcendentals=0,
            bytes_accessed=(M * Ds * 4 + M * N * 4 + N * Dg * 4
                            + Dg * H * 4 + Ds * H * 4 + 2 * H * 4 + B * 4),
        ),
    )(inds2, state_inp, sparse_value_mat, goal_inp, wg, bg2, ws, bs2)

    return sse[0, 0]


# speedup vs baseline: 1.0195x; 1.0175x over previous
import jax
import jax.numpy as jnp
from jax import lax
from jax.experimental import pallas as pl
from jax.experimental.pallas import tpu as pltpu

_TB = 1024
_HT = 128


def _fk(inds_ref,
        state_ref,
        mat_ref,
        goal_ref, wg_ref, bg_ref,
        ws_ref, bs_ref,
        sse_ref,
        w_ref,
        bvec_ref,
        counts_ref,
        accv_ref):
    t = pl.program_id(0)
    nsteps = pl.num_programs(0)
    nhist = (nsteps * _TB) // _HT

    @pl.when(t == 0)
    def _init():
        accv_ref[...] = jnp.zeros_like(accv_ref)
        g = jnp.dot(goal_ref[...].astype(jnp.bfloat16),
                    wg_ref[...].astype(jnp.bfloat16),
                    preferred_element_type=jnp.float32)
        goal_out = (g + bg_ref[...]).astype(jnp.bfloat16)
        w = lax.dot_general(ws_ref[...].astype(jnp.bfloat16), goal_out,
                            (((1,), (1,)), ((), ())),
                            preferred_element_type=jnp.float32)
        w_ref[...] = w.astype(jnp.bfloat16)
        bvec_ref[...] = lax.dot_general(
            bs_ref[...].astype(jnp.bfloat16), goal_out,
            (((1,), (1,)), ((), ())),
            preferred_element_type=jnp.float32)
        idx = inds_ref[...]
        hi = idx // _HT
        lo = idx - hi * _HT
        hi_t = lax.broadcasted_iota(jnp.int32, (nhist, 1), 0)
        lo_t = lax.broadcasted_iota(jnp.int32, (_HT, 1), 0)
        hi_oh = (hi == hi_t).astype(jnp.bfloat16)
        lo_oh = (lo == lo_t).astype(jnp.bfloat16)
        counts = lax.dot_general(
            hi_oh, lo_oh, (((1,), (1,)), ((), ())),
            preferred_element_type=jnp.float32)
        counts_ref[...] = counts.reshape(counts_ref.shape)

    recon = jnp.dot(state_ref[...].astype(jnp.bfloat16), w_ref[...],
                    preferred_element_type=jnp.float32) + bvec_ref[...]
    diff = recon - mat_ref[...]
    sq = diff * diff
    w_row = counts_ref[pl.ds(t, 1), :]
    accv_ref[...] += jnp.dot(w_row, sq,
                             preferred_element_type=jnp.float32)

    @pl.when(t == nsteps - 1)
    def _finalize():
        sse_ref[0, 0] = jnp.sum(accv_ref[...])


@jax.jit
def kernel(inds, state_inp, goal_inp, sparse_value_mat, ws, bs, wg, bg):
    M, Ds = state_inp.shape
    N, Dg = goal_inp.shape
    H = ws.shape[1]
    B = inds.shape[0]

    inds2 = inds.astype(jnp.int32).reshape(1, B)
    bs2 = bs.reshape(1, H).astype(jnp.float32)
    bg2 = bg.reshape(1, H).astype(jnp.float32)

    num_tiles = M // _TB

    sse = pl.pallas_call(
        _fk,
        grid=(num_tiles,),
        in_specs=[
            pl.BlockSpec((1, B), lambda t: (0, 0)),
            pl.BlockSpec((_TB, Ds), lambda t: (t, 0)),
            pl.BlockSpec((_TB, N), lambda t: (t, 0)),
            pl.BlockSpec((N, Dg), lambda t: (0, 0)),
            pl.BlockSpec((Dg, H), lambda t: (0, 0)),
            pl.BlockSpec((1, H), lambda t: (0, 0)),
            pl.BlockSpec((Ds, H), lambda t: (0, 0)),
            pl.BlockSpec((1, H), lambda t: (0, 0)),
        ],
        out_specs=pl.BlockSpec(memory_space=pltpu.MemorySpace.SMEM),
        out_shape=jax.ShapeDtypeStruct((1, 1), jnp.float32),
        scratch_shapes=[
            pltpu.VMEM((Ds, N), jnp.bfloat16),
            pltpu.VMEM((1, N), jnp.float32),
            pltpu.VMEM((M // _TB, _TB), jnp.float32),
            pltpu.VMEM((1, N), jnp.float32),
        ],
        compiler_params=pltpu.CompilerParams(
            dimension_semantics=("arbitrary",),
            vmem_limit_bytes=64 * 1024 * 1024,
        ),
        cost_estimate=pl.CostEstimate(
            flops=2 * N * Dg * H + 2 * Ds * H * N + 2 * M * Ds * N + 3 * M * N,
            transcendentals=0,
            bytes_accessed=(M * Ds * 4 + M * N * 4 + N * Dg * 4
                            + Dg * H * 4 + Ds * H * 4 + 2 * H * 4 + B * 4),
        ),
    )(inds2, state_inp, sparse_value_mat, goal_inp, wg, bg2, ws, bs2)

    return sse[0, 0]
